# baseline (speedup 1.0000x reference)
import functools

import jax
import jax.numpy as jnp
from jax import lax
from jax.experimental import pallas as pl
from jax.experimental.pallas import tpu as pltpu
from jax.experimental.pallas import tpu_sc as plsc

B1 = 3072
BBLK = 256
CB = 2
NW = 32


def _tc_body(a_ref, t_ref, o_ref):
    o_ref[...] = lax.dot_general(
        a_ref[...].astype(jnp.bfloat16), t_ref[...].astype(jnp.bfloat16),
        dimension_numbers=(((0,), (0,)), ((), ())),
        preferred_element_type=jnp.float32,
    )


def _sc_body(b2, at_hbm, tbl_hbm, out_hbm, tbl_v, lut, mask_v, outb):
    c = lax.axis_index("c")
    s = lax.axis_index("s")
    wid = s * 2 + c
    bw = b2 // NW
    b0 = B1 + wid * bw
    pltpu.sync_copy(tbl_hbm, tbl_v)

    zero = jnp.zeros((16,), jnp.float32)
    for g in range(3):
        base = g * 16384
        for dv in range(8):
            lut[pl.ds(base + dv * 16, 16)] = zero
        for j in range(7):
            size = 1 << j

            def _dbl(si, _, base=base, j=j, size=size, g=g):
                src = base + si * 128
                dst = base + (size + si) * 128
                for dv in range(8):
                    tv = tbl_v[7 * g + j, pl.ds(dv * 16, 16)]
                    lut[pl.ds(dst + dv * 16, 16)] = (
                        lut[pl.ds(src + dv * 16, 16)] + tv)
                return 0

            lax.fori_loop(0, size, _dbl, 0)

    lanes = lax.iota(jnp.int32, 16)
    nchunks = bw // CB

    def _chunk(ci, _):
        gb = b0 + ci * CB
        for k in range(21):
            pltpu.sync_copy(at_hbm.at[k, pl.ds(gb, CB), :], mask_v.at[k])
        for bb in range(CB):
            for gg in range(8):
                n0 = gg * 16
                idxs = []
                for g in range(3):
                    acc = jnp.zeros((16,), jnp.float32)
                    for j in range(7):
                        acc = acc + mask_v[7 * g + j, bb, pl.ds(n0, 16)] * float(1 << j)
                    idxs.append(acc.astype(jnp.int32) * 128 + g * 16384)
                stbase = (lanes + (bb * 128 + n0)) * 128

                def _dloop(d, _, idxs=idxs, stbase=stbase):
                    g0 = plsc.load_gather(lut, [idxs[0] + d])
                    g1 = plsc.load_gather(lut, [idxs[1] + d])
                    g2 = plsc.load_gather(lut, [idxs[2] + d])
                    plsc.store_scatter(outb, [stbase + d], g0 + g1 + g2)
                    return 0

                lax.fori_loop(0, 128, _dloop, 0, unroll=4)
        pltpu.sync_copy(outb, out_hbm.at[pl.ds((gb - B1) * 16384, CB * 16384)])
        return 0

    lax.fori_loop(0, nchunks, _chunk, 0)


def _sc_kernel(at, table, b2):
    mesh = plsc.VectorSubcoreMesh(core_axis_name="c", subcore_axis_name="s")
    kfn = pl.kernel(
        functools.partial(_sc_body, b2),
        mesh=mesh,
        out_type=jax.ShapeDtypeStruct((b2 * 16384,), jnp.float32),
        scratch_types=[
            pltpu.VMEM((21, 128), jnp.float32),
            pltpu.VMEM((3 * 128 * 128,), jnp.float32),
            pltpu.VMEM((21, CB, 128), jnp.float32),
            pltpu.VMEM((CB * 128 * 128,), jnp.float32),
        ],
        compiler_params=pltpu.CompilerParams(needs_layout_passes=False),
    )
    return kfn(at, table)


def kernel(alignments, alignment_embeds):
    B, N, K = alignments.shape
    D = alignment_embeds.shape[-1]
    at = jnp.transpose(alignments, (2, 0, 1))
    b2 = B - B1
    out_tc = pl.pallas_call(
        _tc_body,
        grid=(B1 // BBLK,),
        in_specs=[
            pl.BlockSpec((K, BBLK, N), lambda i: (0, i, 0)),
            pl.BlockSpec((K, D), lambda i: (0, 0)),
        ],
        out_specs=pl.BlockSpec((BBLK, N, D), lambda i: (i, 0, 0)),
        out_shape=jax.ShapeDtypeStruct((B, N, D), jnp.float32),
    )(at, alignment_embeds)
    if not b2:
        return out_tc
    out_sc = _sc_kernel(at, alignment_embeds, b2).reshape(b2, N, D)
    return lax.dynamic_update_slice(out_tc, out_sc, (B1, 0, 0))


# speedup vs baseline: 15.8693x; 15.8693x over previous
import jax
import jax.numpy as jnp
from jax.experimental import pallas as pl
from jax.experimental.pallas import tpu as pltpu


def _body(a_ref, t_ref, o_ref):
    o_ref[...] = jax.lax.dot_general(
        a_ref[...].astype(jnp.bfloat16), t_ref[...].astype(jnp.bfloat16),
        dimension_numbers=(((0,), (0,)), ((), ())),
        preferred_element_type=jnp.float32,
    )


def kernel(alignments, alignment_embeds):
    B, N, K = alignments.shape
    D = alignment_embeds.shape[-1]
    BBLK = 256
    at = jnp.transpose(alignments, (2, 0, 1))
    out = pl.pallas_call(
        _body,
        grid=(B // BBLK,),
        in_specs=[
            pl.BlockSpec((K, BBLK, N), lambda i: (0, i, 0)),
            pl.BlockSpec((K, D), lambda i: (0, 0)),
        ],
        out_specs=pl.BlockSpec((BBLK, N, D), lambda i: (i, 0, 0)),
        out_shape=jax.ShapeDtypeStruct((B, N, D), jnp.float32),
        compiler_params=pltpu.CompilerParams(vmem_limit_bytes=100 * 1024 * 1024),
    )(at, alignment_embeds)
    return out
